# baseline (speedup 1.0000x reference)
import jax
import jax.numpy as jnp
from jax import lax
from jax.experimental import pallas as pl
from jax.experimental.pallas import tpu as pltpu
from jax.experimental.pallas import tpu_sc as plsc

N_NODES = 50000
N_PAD = 50048
N_EDGES = 800000
DIM = 64
HALF = 32
NC = 2
NS = 16
CHUNK = 128
KCH = 2
BLK = KCH * CHUNK
NBT = N_EDGES // BLK
E_PAD = NBT * BLK
BASE = NBT // NS
REM = NBT % NS
MAXI = BASE + (1 if REM else 0)
NSE = 4
NSR = 3
UNROLL = 12
STRIPE = N_PAD // NS


def _layer_body(col_hbm, dst_hbm, w_hbm, h_hbm, out_hbm, *refs):
    cbufs = refs[0:NSE]
    dbufs = refs[NSE:2 * NSE]
    wbufs = refs[2 * NSE:3 * NSE]
    rbufs = tuple(tuple(refs[3 * NSE + u * KCH + k] for k in range(KCH))
                  for u in range(NSR))
    acc_sh, semA, semG, semS = refs[3 * NSE + NSR * KCH:]
    c = lax.axis_index("c")
    s = lax.axis_index("s")
    n_mine = BASE + (s < REM).astype(jnp.int32)

    def blk_id(j):
        return s + j * NS

    def linear_trio(j, u):
        b = blk_id(j)
        return (pltpu.make_async_copy(col_hbm.at[pl.ds(b * BLK, BLK)], cbufs[u],
                                      semA.at[u]),
                pltpu.make_async_copy(dst_hbm.at[b], dbufs[u], semA.at[u]),
                pltpu.make_async_copy(w_hbm.at[b], wbufs[u], semA.at[u]))

    def gather_desc(ue, ur, k):
        idx = cbufs[ue].at[pl.ds(k * CHUNK, CHUNK)]
        return pltpu.make_async_copy(h_hbm.at[c].at[idx], rbufs[ur][k],
                                     semG.at[ur])

    def scatter_desc(ue, ur, k):
        return pltpu.make_async_copy(rbufs[ur][k], acc_sh.at[dbufs[ue].at[k]],
                                     semS.at[ur])

    def A(j, u):
        for d in linear_trio(j, u):
            d.start()

    def M(j, ue, ur):
        for d in linear_trio(j, ue):
            d.wait()
        for k in range(KCH):
            gather_desc(ue, ur, k).start()

    def F(ue, ur):
        for k in range(KCH):
            gather_desc(ue, ur, k).wait()
        wbuf = wbufs[ue]

        def multq(q, _):
            rr = q * 16
            for k in range(KCH):
                rows = rbufs[ur][k]
                wv = wbuf[k, pl.ds(rr, 16)]
                for jj in range(16):
                    w = wv[jj]
                    rows[rr + jj, pl.ds(0, 16)] = rows[rr + jj, pl.ds(0, 16)] * w
                    rows[rr + jj, pl.ds(16, 16)] = rows[rr + jj, pl.ds(16, 16)] * w
            return 0
        lax.fori_loop(0, 8, multq, 0)
        for k in range(KCH):
            scatter_desc(ue, ur, k).start(add=True)

    def D(ue, ur):
        for k in range(KCH):
            scatter_desc(ue, ur, k).wait()

    for j in range(3):
        A(jnp.int32(j), j)

    zeros16 = jnp.zeros((16,), jnp.float32)

    zref = rbufs[0][0]

    def zfill(jj, _):
        zref[jj, pl.ds(0, 16)] = zeros16
        zref[jj, pl.ds(16, 16)] = zeros16
        return 0
    lax.fori_loop(0, CHUNK, zfill, 0)

    zbase = s * STRIPE

    def zacc(m, _):
        pltpu.sync_copy(zref, acc_sh.at[pl.ds(zbase + m * CHUNK, CHUNK)])
        return 0
    lax.fori_loop(0, STRIPE // CHUNK, zacc, 0)
    pltpu.sync_copy(zref.at[pl.ds(0, STRIPE % CHUNK)],
                    acc_sh.at[pl.ds(zbase + (STRIPE // CHUNK) * CHUNK, STRIPE % CHUNK)])
    plsc.subcore_barrier()

    M(jnp.int32(0), 0, 0)
    M(jnp.int32(1), 1, 1)

    NSTEP = -(-(MAXI + 1) // UNROLL)

    def step(t, _):
        for u in range(UNROLL):
            i = t * UNROLL + u

            @pl.when(i < n_mine)
            def _(u=u):
                F(u % NSE, u % NSR)

            @pl.when(jnp.logical_and(i >= 1, i - 1 < n_mine))
            def _(u=u):
                D((u - 1) % NSE, (u - 1) % NSR)

            @pl.when(i + 2 < n_mine)
            def _(i=i, u=u):
                M(i + 2, (u + 2) % NSE, (u + 2) % NSR)

            @pl.when(i + 3 < n_mine)
            def _(i=i, u=u):
                A(i + 3, (u + 3) % NSE)
        return 0
    lax.fori_loop(0, NSTEP, step, 0)

    plsc.subcore_barrier()
    pltpu.sync_copy(acc_sh.at[pl.ds(s * STRIPE, STRIPE)],
                    out_hbm.at[c].at[pl.ds(s * STRIPE, STRIPE)])


@jax.jit
def _layer(col, dst, w, h):
    return pl.kernel(
        _layer_body,
        out_type=jax.ShapeDtypeStruct((NC, N_PAD, HALF), jnp.float32),
        mesh=plsc.VectorSubcoreMesh(core_axis_name="c", subcore_axis_name="s"),
        scratch_types=(
            [pltpu.VMEM((BLK,), jnp.int32) for _ in range(NSE)]
            + [pltpu.VMEM((KCH, CHUNK), jnp.int32) for _ in range(NSE)]
            + [pltpu.VMEM((KCH, CHUNK), jnp.float32) for _ in range(NSE)]
            + [pltpu.VMEM((CHUNK, HALF), jnp.float32) for _ in range(NSR * KCH)]
            + [
                pltpu.VMEM_SHARED((N_PAD, HALF), jnp.float32),
                pltpu.SemaphoreType.DMA((NSE,)),
                pltpu.SemaphoreType.DMA((NSR,)),
                pltpu.SemaphoreType.DMA((NSR,)),
            ]
        ),
        compiler_params=pltpu.CompilerParams(use_tc_tiling_on_sc=False),
    )(col, dst, w, h)


USER_N = 10000
SB = 2000


def _split_body(u_ref, it_ref, o_ref):
    i = pl.program_id(0)

    def write(x):
        o_ref[0] = x[:, :HALF]
        o_ref[1] = x[:, HALF:]

    @pl.when(i < USER_N // SB)
    def _():
        write(u_ref[...])

    @pl.when(i >= USER_N // SB)
    def _():
        write(it_ref[...])


@jax.jit
def _split(user_emb, item_emb):
    nu = USER_N // SB
    out = pl.pallas_call(
        _split_body,
        out_shape=jax.ShapeDtypeStruct((NC, N_PAD, HALF), jnp.float32),
        grid=(N_NODES // SB,),
        in_specs=[
            pl.BlockSpec((SB, DIM), lambda i: (jnp.minimum(i, nu - 1), 0)),
            pl.BlockSpec((SB, DIM), lambda i: (jnp.maximum(i - nu, 0), 0)),
        ],
        out_specs=pl.BlockSpec((NC, SB, HALF), lambda i: (0, i, 0)),
    )(user_emb, item_emb)
    return out


QROWS = N_NODES // 4
CB = 544


def _combine_body(a, b, e, o):
    x0 = (a[0] + b[0] + e[0]) * (1.0 / 3.0)
    x1 = (a[1] + b[1] + e[1]) * (1.0 / 3.0)
    parts = [jnp.concatenate([x0[:, 32 * t:32 * t + 32], x1[:, 32 * t:32 * t + 32]],
                             axis=1) for t in range(4)]
    y = jnp.stack(parts, axis=1)
    o[...] = y.reshape(4 * CB, DIM)


@jax.jit
def _combine(h0, h1, h2):
    spec = pl.BlockSpec((2, CB, 128), lambda i: (0, i, 0))
    return pl.pallas_call(
        _combine_body,
        out_shape=jax.ShapeDtypeStruct((N_NODES, DIM), jnp.float32),
        grid=(-(-QROWS // CB),),
        in_specs=[spec, spec, spec],
        out_specs=pl.BlockSpec((4 * CB, DIM), lambda i: (i, 0)),
    )(h0.reshape(NC, N_PAD // 4, 128), h1.reshape(NC, N_PAD // 4, 128),
      h2.reshape(NC, N_PAD // 4, 128))


def kernel(edge_index, edge_weight, user_emb, item_emb):
    dst = edge_index[0].astype(jnp.int32).reshape(NBT, KCH, CHUNK)
    col = edge_index[1].astype(jnp.int32)
    ew = edge_weight.reshape(NBT, KCH, CHUNK)

    h0 = _split(user_emb, item_emb)
    h1 = _layer(col, dst, ew, h0)
    h2 = _layer(col, dst, ew, h1)
    return _combine(h0, h1, h2)


# speedup vs baseline: 1.0969x; 1.0969x over previous
import jax
import jax.numpy as jnp
from jax import lax
from jax.experimental import pallas as pl
from jax.experimental.pallas import tpu as pltpu
from jax.experimental.pallas import tpu_sc as plsc

N_NODES = 50000
N_PAD = 50048
N_EDGES = 800000
DIM = 64
HALF = 32
NC = 2
NS = 16
CHUNK = 128
KCH = 2
BLK = KCH * CHUNK
NBT = N_EDGES // BLK
E_PAD = NBT * BLK
BASE = NBT // NS
REM = NBT % NS
MAXI = BASE + (1 if REM else 0)
NSE = 4
NSR = 3
UNROLL = 12
STRIPE = N_PAD // NS


def _layer_body(col_hbm, dst_hbm, w_hbm, h_hbm, out_hbm, *refs):
    cbufs = refs[0:NSE]
    dbufs = refs[NSE:2 * NSE]
    wbufs = refs[2 * NSE:3 * NSE]
    rbufs = tuple(tuple(refs[3 * NSE + u * KCH + k] for k in range(KCH))
                  for u in range(NSR))
    acc_sh, semA, semG, semS = refs[3 * NSE + NSR * KCH:]
    c = lax.axis_index("c")
    s = lax.axis_index("s")
    n_mine = BASE + (s < REM).astype(jnp.int32)

    def blk_id(j):
        return s + j * NS

    def linear_trio(j, u):
        b = blk_id(j)
        return (pltpu.make_async_copy(col_hbm.at[pl.ds(b * BLK, BLK)], cbufs[u],
                                      semA.at[u]),
                pltpu.make_async_copy(dst_hbm.at[b], dbufs[u], semA.at[u]),
                pltpu.make_async_copy(w_hbm.at[b], wbufs[u], semA.at[u]))

    def gather_desc(ue, ur, k):
        idx = cbufs[ue].at[pl.ds(k * CHUNK, CHUNK)]
        return pltpu.make_async_copy(h_hbm.at[c].at[idx], rbufs[ur][k],
                                     semG.at[ur])

    def scatter_desc(ue, ur, k):
        return pltpu.make_async_copy(rbufs[ur][k], acc_sh.at[dbufs[ue].at[k]],
                                     semS.at[ur])

    def A(j, u):
        for d in linear_trio(j, u):
            d.start()

    def M(j, ue, ur):
        for d in linear_trio(j, ue):
            d.wait()
        for k in range(KCH):
            gather_desc(ue, ur, k).start()

    def F(ue, ur):
        for k in range(KCH):
            gather_desc(ue, ur, k).wait()
        wbuf = wbufs[ue]

        def multq(q, _):
            rr = q * 16
            for k in range(KCH):
                rows = rbufs[ur][k]
                wv = wbuf[k, pl.ds(rr, 16)]
                for jj in range(16):
                    w = wv[jj]
                    rows[rr + jj, pl.ds(0, 16)] = rows[rr + jj, pl.ds(0, 16)] * w
                    rows[rr + jj, pl.ds(16, 16)] = rows[rr + jj, pl.ds(16, 16)] * w
            return 0
        lax.fori_loop(0, 8, multq, 0)
        for k in range(KCH):
            scatter_desc(ue, ur, k).start(add=True)

    def D(ue, ur):
        for k in range(KCH):
            scatter_desc(ue, ur, k).wait()

    for j in range(3):
        A(jnp.int32(j), j)

    zeros16 = jnp.zeros((16,), jnp.float32)

    zref = rbufs[0][0]

    def zfill(jj, _):
        zref[jj, pl.ds(0, 16)] = zeros16
        zref[jj, pl.ds(16, 16)] = zeros16
        return 0
    lax.fori_loop(0, CHUNK, zfill, 0)

    zbase = s * STRIPE

    def zacc(m, _):
        pltpu.sync_copy(zref, acc_sh.at[pl.ds(zbase + m * CHUNK, CHUNK)])
        return 0
    lax.fori_loop(0, STRIPE // CHUNK, zacc, 0)
    pltpu.sync_copy(zref.at[pl.ds(0, STRIPE % CHUNK)],
                    acc_sh.at[pl.ds(zbase + (STRIPE // CHUNK) * CHUNK, STRIPE % CHUNK)])
    plsc.subcore_barrier()

    M(jnp.int32(0), 0, 0)
    M(jnp.int32(1), 1, 1)

    NSTEP = -(-(MAXI + 1) // UNROLL)

    def step(t, _):
        for u in range(UNROLL):
            i = t * UNROLL + u

            @pl.when(i < n_mine)
            def _(u=u):
                F(u % NSE, u % NSR)

            @pl.when(jnp.logical_and(i >= 1, i - 1 < n_mine))
            def _(u=u):
                D((u - 1) % NSE, (u - 1) % NSR)

            @pl.when(i + 2 < n_mine)
            def _(i=i, u=u):
                M(i + 2, (u + 2) % NSE, (u + 2) % NSR)

            @pl.when(i + 3 < n_mine)
            def _(i=i, u=u):
                A(i + 3, (u + 3) % NSE)
        return 0
    lax.fori_loop(0, NSTEP, step, 0)

    plsc.subcore_barrier()
    pltpu.sync_copy(acc_sh.at[pl.ds(s * STRIPE, STRIPE)],
                    out_hbm.at[c].at[pl.ds(s * STRIPE, STRIPE)])


@jax.jit
def _layer(col, dst, w, h):
    return pl.kernel(
        _layer_body,
        out_type=jax.ShapeDtypeStruct((NC, N_PAD, HALF), jnp.float32),
        mesh=plsc.VectorSubcoreMesh(core_axis_name="c", subcore_axis_name="s"),
        scratch_types=(
            [pltpu.VMEM((BLK,), jnp.int32) for _ in range(NSE)]
            + [pltpu.VMEM((KCH, CHUNK), jnp.int32) for _ in range(NSE)]
            + [pltpu.VMEM((KCH, CHUNK), jnp.float32) for _ in range(NSE)]
            + [pltpu.VMEM((CHUNK, HALF), jnp.float32) for _ in range(NSR * KCH)]
            + [
                pltpu.VMEM_SHARED((N_PAD, HALF), jnp.float32),
                pltpu.SemaphoreType.DMA((NSE,)),
                pltpu.SemaphoreType.DMA((NSR,)),
                pltpu.SemaphoreType.DMA((NSR,)),
            ]
        ),
        compiler_params=pltpu.CompilerParams(use_tc_tiling_on_sc=False),
    )(col, dst, w, h)


USER_N = 10000


def _split_body(user_hbm, item_hbm, h0_hbm, b0, b1, bx, sem):
    c = lax.axis_index("c")
    s = lax.axis_index("s")
    bufs = (b0, b1)
    zbase = s * STRIPE

    def start_of(m):
        return jnp.minimum(zbase + m * CHUNK, N_NODES - CHUNK)

    def descs(m, u):
        start = start_of(m)
        si = jnp.minimum(jnp.maximum(start, USER_N), N_NODES - CHUNK)
        return (
            start + CHUNK <= USER_N,
            pltpu.make_async_copy(user_hbm.at[pl.ds(start, CHUNK)], bufs[u],
                                  sem.at[u]),
            pltpu.make_async_copy(item_hbm.at[pl.ds(si - USER_N, CHUNK)], bufs[u],
                                  sem.at[u]),
            si,
        )

    def issue(m, u):
        is_user, du, di, _ = descs(m, u)

        @pl.when(is_user)
        def _():
            du.start()

        @pl.when(jnp.logical_not(is_user))
        def _():
            di.start()

    NCH = STRIPE // CHUNK + 1
    issue(jnp.int32(0), 0)

    def step(t, _):
        for u in range(2):
            m = t * 2 + u

            @pl.when(m < NCH)
            def _(m=m, u=u):
                @pl.when(m + 1 < NCH)
                def _():
                    issue(m + 1, 1 - u)
                is_user, du, _, si = descs(m, u)
                du.wait()
                start = start_of(m)
                wstart = jnp.where(is_user, start, si)
                pltpu.sync_copy(bufs[u].at[:, pl.ds(c * HALF, HALF)],
                                h0_hbm.at[c].at[pl.ds(wstart, CHUNK)])
                straddle = jnp.logical_and(start < USER_N, start + CHUNK > USER_N)

                @pl.when(straddle)
                def _():
                    pltpu.sync_copy(user_hbm.at[pl.ds(USER_N - CHUNK, CHUNK)], bx)
                    pltpu.sync_copy(bx.at[:, pl.ds(c * HALF, HALF)],
                                    h0_hbm.at[c].at[pl.ds(USER_N - CHUNK, CHUNK)])
        return 0
    lax.fori_loop(0, (NCH + 1) // 2, step, 0)


@jax.jit
def _split(user_emb, item_emb):
    return pl.kernel(
        _split_body,
        out_type=jax.ShapeDtypeStruct((NC, N_PAD, HALF), jnp.float32),
        mesh=plsc.VectorSubcoreMesh(core_axis_name="c", subcore_axis_name="s"),
        scratch_types=[
            pltpu.VMEM((CHUNK, DIM), jnp.float32),
            pltpu.VMEM((CHUNK, DIM), jnp.float32),
            pltpu.VMEM((CHUNK, DIM), jnp.float32),
            pltpu.SemaphoreType.DMA((2,)),
        ],
        compiler_params=pltpu.CompilerParams(use_tc_tiling_on_sc=False),
    )(user_emb, item_emb)


QROWS = N_NODES // 4
CB = 544


def _combine_body(a, b, e, o):
    x0 = (a[0] + b[0] + e[0]) * (1.0 / 3.0)
    x1 = (a[1] + b[1] + e[1]) * (1.0 / 3.0)
    o[:, 0, :] = jnp.concatenate(
        [x0[:, 0:32], x1[:, 0:32], x0[:, 32:64], x1[:, 32:64]], axis=1)
    o[:, 1, :] = jnp.concatenate(
        [x0[:, 64:96], x1[:, 64:96], x0[:, 96:128], x1[:, 96:128]], axis=1)


@jax.jit
def _combine(h0, h1, h2):
    spec = pl.BlockSpec((2, CB, 128), lambda i: (0, i, 0))
    out = pl.pallas_call(
        _combine_body,
        out_shape=jax.ShapeDtypeStruct((QROWS, 2, 128), jnp.float32),
        grid=(-(-QROWS // CB),),
        in_specs=[spec, spec, spec],
        out_specs=pl.BlockSpec((CB, 2, 128), lambda i: (i, 0, 0)),
    )(h0.reshape(NC, N_PAD // 4, 128), h1.reshape(NC, N_PAD // 4, 128),
      h2.reshape(NC, N_PAD // 4, 128))
    return out.reshape(N_NODES, DIM)


def kernel(edge_index, edge_weight, user_emb, item_emb):
    dst = edge_index[0].astype(jnp.int32).reshape(NBT, KCH, CHUNK)
    col = edge_index[1].astype(jnp.int32)
    ew = edge_weight.reshape(NBT, KCH, CHUNK)

    h0 = _split(user_emb, item_emb)
    h1 = _layer(col, dst, ew, h0)
    h2 = _layer(col, dst, ew, h1)
    return _combine(h0, h1, h2)


# speedup vs baseline: 1.0995x; 1.0023x over previous
import jax
import jax.numpy as jnp
from jax import lax
from jax.experimental import pallas as pl
from jax.experimental.pallas import tpu as pltpu
from jax.experimental.pallas import tpu_sc as plsc

N_NODES = 50000
N_PAD = 50048
N_EDGES = 800000
DIM = 64
HALF = 32
NC = 2
NS = 16
CHUNK = 128
KCH = 2
BLK = KCH * CHUNK
NBT = N_EDGES // BLK
E_PAD = NBT * BLK
BASE = NBT // NS
REM = NBT % NS
MAXI = BASE + (1 if REM else 0)
NSE = 4
NSR = 3
UNROLL = 12
STRIPE = N_PAD // NS


def _layer_body(col_hbm, dst_hbm, w_hbm, h_hbm, out_hbm, *refs):
    cbufs = refs[0:NSE]
    dbufs = refs[NSE:2 * NSE]
    wbufs = refs[2 * NSE:3 * NSE]
    rbufs = tuple(tuple(refs[3 * NSE + u * KCH + k] for k in range(KCH))
                  for u in range(NSR))
    acc_sh, semA, semG, semS = refs[3 * NSE + NSR * KCH:]
    c = lax.axis_index("c")
    s = lax.axis_index("s")
    n_mine = BASE + (s < REM).astype(jnp.int32)

    def blk_id(j):
        return s + j * NS

    def linear_trio(j, u):
        b = blk_id(j)
        return (pltpu.make_async_copy(col_hbm.at[pl.ds(b * BLK, BLK)], cbufs[u],
                                      semA.at[u]),
                pltpu.make_async_copy(dst_hbm.at[b], dbufs[u], semA.at[u]),
                pltpu.make_async_copy(w_hbm.at[b], wbufs[u], semA.at[u]))

    def gather_desc(ue, ur, k):
        idx = cbufs[ue].at[pl.ds(k * CHUNK, CHUNK)]
        return pltpu.make_async_copy(h_hbm.at[c].at[idx], rbufs[ur][k],
                                     semG.at[ur])

    def scatter_desc(ue, ur, k):
        return pltpu.make_async_copy(rbufs[ur][k], acc_sh.at[dbufs[ue].at[k]],
                                     semS.at[ur])

    def A(j, u):
        for d in linear_trio(j, u):
            d.start()

    def M(j, ue, ur):
        for d in linear_trio(j, ue):
            d.wait()
        for k in range(KCH):
            gather_desc(ue, ur, k).start()

    def F(ue, ur):
        wbuf = wbufs[ue]
        for k in range(KCH):
            gather_desc(ue, ur, k).wait()
            rows = rbufs[ur][k]

            def multq(q, _, k=k, rows=rows):
                rr = q * 16
                wv = wbuf[k, pl.ds(rr, 16)]
                for jj in range(16):
                    w = wv[jj]
                    rows[rr + jj, pl.ds(0, 16)] = rows[rr + jj, pl.ds(0, 16)] * w
                    rows[rr + jj, pl.ds(16, 16)] = rows[rr + jj, pl.ds(16, 16)] * w
                return 0
            lax.fori_loop(0, 8, multq, 0)
            scatter_desc(ue, ur, k).start(add=True)

    def D(ue, ur):
        for k in range(KCH):
            scatter_desc(ue, ur, k).wait()

    for j in range(3):
        A(jnp.int32(j), j)
    M(jnp.int32(0), 0, 0)
    M(jnp.int32(1), 1, 1)

    zeros16 = jnp.zeros((16,), jnp.float32)

    zref = rbufs[2][0]

    def zfill(jj, _):
        zref[jj, pl.ds(0, 16)] = zeros16
        zref[jj, pl.ds(16, 16)] = zeros16
        return 0
    lax.fori_loop(0, CHUNK, zfill, 0)

    zbase = s * STRIPE
    ZN = STRIPE // CHUNK

    def zdesc(m):
        return pltpu.make_async_copy(zref, acc_sh.at[pl.ds(zbase + m * CHUNK, CHUNK)],
                                     semS.at[0])

    ztail = pltpu.make_async_copy(
        zref.at[pl.ds(0, STRIPE % CHUNK)],
        acc_sh.at[pl.ds(zbase + ZN * CHUNK, STRIPE % CHUNK)], semS.at[0])

    def zacc(m, _):
        zdesc(m).start()
        return 0
    lax.fori_loop(0, ZN, zacc, 0)
    ztail.start()

    def zwait(m, _):
        zdesc(m).wait()
        return 0
    lax.fori_loop(0, ZN, zwait, 0)
    ztail.wait()
    plsc.subcore_barrier()

    NSTEP = -(-(MAXI + 1) // UNROLL)

    def step(t, _):
        for u in range(UNROLL):
            i = t * UNROLL + u

            @pl.when(i < n_mine)
            def _(u=u):
                F(u % NSE, u % NSR)

            @pl.when(jnp.logical_and(i >= 1, i - 1 < n_mine))
            def _(u=u):
                D((u - 1) % NSE, (u - 1) % NSR)

            @pl.when(i + 2 < n_mine)
            def _(i=i, u=u):
                M(i + 2, (u + 2) % NSE, (u + 2) % NSR)

            @pl.when(i + 3 < n_mine)
            def _(i=i, u=u):
                A(i + 3, (u + 3) % NSE)
        return 0
    lax.fori_loop(0, NSTEP, step, 0)

    plsc.subcore_barrier()
    pltpu.sync_copy(acc_sh.at[pl.ds(s * STRIPE, STRIPE)],
                    out_hbm.at[c].at[pl.ds(s * STRIPE, STRIPE)])


@jax.jit
def _layer(col, dst, w, h):
    return pl.kernel(
        _layer_body,
        out_type=jax.ShapeDtypeStruct((NC, N_PAD, HALF), jnp.float32),
        mesh=plsc.VectorSubcoreMesh(core_axis_name="c", subcore_axis_name="s"),
        scratch_types=(
            [pltpu.VMEM((BLK,), jnp.int32) for _ in range(NSE)]
            + [pltpu.VMEM((KCH, CHUNK), jnp.int32) for _ in range(NSE)]
            + [pltpu.VMEM((KCH, CHUNK), jnp.float32) for _ in range(NSE)]
            + [pltpu.VMEM((CHUNK, HALF), jnp.float32) for _ in range(NSR * KCH)]
            + [
                pltpu.VMEM_SHARED((N_PAD, HALF), jnp.float32),
                pltpu.SemaphoreType.DMA((NSE,)),
                pltpu.SemaphoreType.DMA((NSR,)),
                pltpu.SemaphoreType.DMA((NSR,)),
            ]
        ),
        compiler_params=pltpu.CompilerParams(use_tc_tiling_on_sc=False),
    )(col, dst, w, h)


USER_N = 10000


def _split_body(user_hbm, item_hbm, h0_hbm, b0, b1, bx, sem):
    c = lax.axis_index("c")
    s = lax.axis_index("s")
    bufs = (b0, b1)
    zbase = s * STRIPE

    def start_of(m):
        return jnp.minimum(zbase + m * CHUNK, N_NODES - CHUNK)

    def descs(m, u):
        start = start_of(m)
        si = jnp.minimum(jnp.maximum(start, USER_N), N_NODES - CHUNK)
        return (
            start + CHUNK <= USER_N,
            pltpu.make_async_copy(user_hbm.at[pl.ds(start, CHUNK)], bufs[u],
                                  sem.at[u]),
            pltpu.make_async_copy(item_hbm.at[pl.ds(si - USER_N, CHUNK)], bufs[u],
                                  sem.at[u]),
            si,
        )

    def issue(m, u):
        is_user, du, di, _ = descs(m, u)

        @pl.when(is_user)
        def _():
            du.start()

        @pl.when(jnp.logical_not(is_user))
        def _():
            di.start()

    NCH = STRIPE // CHUNK + 1
    issue(jnp.int32(0), 0)

    def step(t, _):
        for u in range(2):
            m = t * 2 + u

            @pl.when(m < NCH)
            def _(m=m, u=u):
                @pl.when(m + 1 < NCH)
                def _():
                    issue(m + 1, 1 - u)
                is_user, du, _, si = descs(m, u)
                du.wait()
                start = start_of(m)
                wstart = jnp.where(is_user, start, si)
                pltpu.sync_copy(bufs[u].at[:, pl.ds(c * HALF, HALF)],
                                h0_hbm.at[c].at[pl.ds(wstart, CHUNK)])
                straddle = jnp.logical_and(start < USER_N, start + CHUNK > USER_N)

                @pl.when(straddle)
                def _():
                    pltpu.sync_copy(user_hbm.at[pl.ds(USER_N - CHUNK, CHUNK)], bx)
                    pltpu.sync_copy(bx.at[:, pl.ds(c * HALF, HALF)],
                                    h0_hbm.at[c].at[pl.ds(USER_N - CHUNK, CHUNK)])
        return 0
    lax.fori_loop(0, (NCH + 1) // 2, step, 0)


@jax.jit
def _split(user_emb, item_emb):
    return pl.kernel(
        _split_body,
        out_type=jax.ShapeDtypeStruct((NC, N_PAD, HALF), jnp.float32),
        mesh=plsc.VectorSubcoreMesh(core_axis_name="c", subcore_axis_name="s"),
        scratch_types=[
            pltpu.VMEM((CHUNK, DIM), jnp.float32),
            pltpu.VMEM((CHUNK, DIM), jnp.float32),
            pltpu.VMEM((CHUNK, DIM), jnp.float32),
            pltpu.SemaphoreType.DMA((2,)),
        ],
        compiler_params=pltpu.CompilerParams(use_tc_tiling_on_sc=False),
    )(user_emb, item_emb)


QROWS = N_NODES // 4
CB = 544


def _combine_body(a, b, e, o):
    x0 = (a[0] + b[0] + e[0]) * (1.0 / 3.0)
    x1 = (a[1] + b[1] + e[1]) * (1.0 / 3.0)
    o[:, 0, :] = jnp.concatenate(
        [x0[:, 0:32], x1[:, 0:32], x0[:, 32:64], x1[:, 32:64]], axis=1)
    o[:, 1, :] = jnp.concatenate(
        [x0[:, 64:96], x1[:, 64:96], x0[:, 96:128], x1[:, 96:128]], axis=1)


@jax.jit
def _combine(h0, h1, h2):
    spec = pl.BlockSpec((2, CB, 128), lambda i: (0, i, 0))
    out = pl.pallas_call(
        _combine_body,
        out_shape=jax.ShapeDtypeStruct((QROWS, 2, 128), jnp.float32),
        grid=(-(-QROWS // CB),),
        in_specs=[spec, spec, spec],
        out_specs=pl.BlockSpec((CB, 2, 128), lambda i: (i, 0, 0)),
    )(h0.reshape(NC, N_PAD // 4, 128), h1.reshape(NC, N_PAD // 4, 128),
      h2.reshape(NC, N_PAD // 4, 128))
    return out.reshape(N_NODES, DIM)


def kernel(edge_index, edge_weight, user_emb, item_emb):
    dst = edge_index[0].astype(jnp.int32).reshape(NBT, KCH, CHUNK)
    col = edge_index[1].astype(jnp.int32)
    ew = edge_weight.reshape(NBT, KCH, CHUNK)

    h0 = _split(user_emb, item_emb)
    h1 = _layer(col, dst, ew, h0)
    h2 = _layer(col, dst, ew, h1)
    return _combine(h0, h1, h2)


# speedup vs baseline: 1.1157x; 1.0148x over previous
import jax
import jax.numpy as jnp
from jax import lax
from jax.experimental import pallas as pl
from jax.experimental.pallas import tpu as pltpu
from jax.experimental.pallas import tpu_sc as plsc

N_NODES = 50000
N_PAD = 50048
N_EDGES = 800000
DIM = 64
HALF = 32
NC = 2
NS = 16
CHUNK = 128
KCH = 2
BLK = KCH * CHUNK
NBT = N_EDGES // BLK
E_PAD = NBT * BLK
BASE = NBT // NS
REM = NBT % NS
MAXI = BASE + (1 if REM else 0)
NSE = 4
NSR = 3
UNROLL = 12
STRIPE = N_PAD // NS


def _layer_body(ei_hbm, w_hbm, h_hbm, out_hbm, *refs):
    cbufs = refs[0:NSE]
    dbufs = refs[NSE:2 * NSE]
    wbufs = refs[2 * NSE:3 * NSE]
    rbufs = tuple(tuple(refs[3 * NSE + u * KCH + k] for k in range(KCH))
                  for u in range(NSR))
    acc_sh, semA, semG, semS = refs[3 * NSE + NSR * KCH:]
    c = lax.axis_index("c")
    s = lax.axis_index("s")
    n_mine = BASE + (s < REM).astype(jnp.int32)

    def blk_id(j):
        return s + j * NS

    def linear_trio(j, u):
        b = blk_id(j)
        return (pltpu.make_async_copy(ei_hbm.at[1].at[b], cbufs[u], semA.at[u]),
                pltpu.make_async_copy(ei_hbm.at[0].at[b], dbufs[u], semA.at[u]),
                pltpu.make_async_copy(w_hbm.at[pl.ds(b * BLK, BLK)], wbufs[u],
                                      semA.at[u]))

    def gather_desc(ue, ur, k):
        return pltpu.make_async_copy(h_hbm.at[c].at[cbufs[ue].at[k]], rbufs[ur][k],
                                     semG.at[ur])

    def scatter_desc(ue, ur, k):
        return pltpu.make_async_copy(rbufs[ur][k], acc_sh.at[dbufs[ue].at[k]],
                                     semS.at[ur])

    def A(j, u):
        for d in linear_trio(j, u):
            d.start()

    def M(j, ue, ur):
        for d in linear_trio(j, ue):
            d.wait()
        for k in range(KCH):
            gather_desc(ue, ur, k).start()

    def F(ue, ur):
        wbuf = wbufs[ue]
        for k in range(KCH):
            gather_desc(ue, ur, k).wait()
            rows = rbufs[ur][k]

            def multq(q, _, k=k, rows=rows):
                rr = q * 16
                wv = wbuf[pl.ds(k * CHUNK + rr, 16)]
                for jj in range(16):
                    w = wv[jj]
                    rows[rr + jj, pl.ds(0, 16)] = rows[rr + jj, pl.ds(0, 16)] * w
                    rows[rr + jj, pl.ds(16, 16)] = rows[rr + jj, pl.ds(16, 16)] * w
                return 0
            lax.fori_loop(0, 8, multq, 0)
            scatter_desc(ue, ur, k).start(add=True)

    def D(ue, ur):
        for k in range(KCH):
            scatter_desc(ue, ur, k).wait()

    for j in range(3):
        A(jnp.int32(j), j)
    M(jnp.int32(0), 0, 0)
    M(jnp.int32(1), 1, 1)

    zeros16 = jnp.zeros((16,), jnp.float32)

    zref = rbufs[2][0]

    def zfill(jj, _):
        zref[jj, pl.ds(0, 16)] = zeros16
        zref[jj, pl.ds(16, 16)] = zeros16
        return 0
    lax.fori_loop(0, CHUNK, zfill, 0)

    zbase = s * STRIPE
    ZN = STRIPE // CHUNK

    def zdesc(m):
        return pltpu.make_async_copy(zref, acc_sh.at[pl.ds(zbase + m * CHUNK, CHUNK)],
                                     semS.at[0])

    ztail = pltpu.make_async_copy(
        zref.at[pl.ds(0, STRIPE % CHUNK)],
        acc_sh.at[pl.ds(zbase + ZN * CHUNK, STRIPE % CHUNK)], semS.at[0])

    def zacc(m, _):
        zdesc(m).start()
        return 0
    lax.fori_loop(0, ZN, zacc, 0)
    ztail.start()

    def zwait(m, _):
        zdesc(m).wait()
        return 0
    lax.fori_loop(0, ZN, zwait, 0)
    ztail.wait()
    plsc.subcore_barrier()

    NSTEP = -(-(MAXI + 1) // UNROLL)

    def step(t, _):
        for u in range(UNROLL):
            i = t * UNROLL + u

            @pl.when(i < n_mine)
            def _(u=u):
                F(u % NSE, u % NSR)

            @pl.when(jnp.logical_and(i >= 1, i - 1 < n_mine))
            def _(u=u):
                D((u - 1) % NSE, (u - 1) % NSR)

            @pl.when(i + 2 < n_mine)
            def _(i=i, u=u):
                M(i + 2, (u + 2) % NSE, (u + 2) % NSR)

            @pl.when(i + 3 < n_mine)
            def _(i=i, u=u):
                A(i + 3, (u + 3) % NSE)
        return 0
    lax.fori_loop(0, NSTEP, step, 0)

    plsc.subcore_barrier()
    pltpu.sync_copy(acc_sh.at[pl.ds(s * STRIPE, STRIPE)],
                    out_hbm.at[c].at[pl.ds(s * STRIPE, STRIPE)])


@jax.jit
def _layer(ei, w, h):
    return pl.kernel(
        _layer_body,
        out_type=jax.ShapeDtypeStruct((NC, N_PAD, HALF), jnp.float32),
        mesh=plsc.VectorSubcoreMesh(core_axis_name="c", subcore_axis_name="s"),
        scratch_types=(
            [pltpu.VMEM((KCH, CHUNK), jnp.int32) for _ in range(NSE)]
            + [pltpu.VMEM((KCH, CHUNK), jnp.int32) for _ in range(NSE)]
            + [pltpu.VMEM((BLK,), jnp.float32) for _ in range(NSE)]
            + [pltpu.VMEM((CHUNK, HALF), jnp.float32) for _ in range(NSR * KCH)]
            + [
                pltpu.VMEM_SHARED((N_PAD, HALF), jnp.float32),
                pltpu.SemaphoreType.DMA((NSE,)),
                pltpu.SemaphoreType.DMA((NSR,)),
                pltpu.SemaphoreType.DMA((NSR,)),
            ]
        ),
        compiler_params=pltpu.CompilerParams(use_tc_tiling_on_sc=False),
    )(ei, w, h)


USER_N = 10000


def _split_body(user_hbm, item_hbm, h0_hbm, b0, b1, bx, sem):
    c = lax.axis_index("c")
    s = lax.axis_index("s")
    bufs = (b0, b1)
    zbase = s * STRIPE

    def start_of(m):
        return jnp.minimum(zbase + m * CHUNK, N_NODES - CHUNK)

    def descs(m, u):
        start = start_of(m)
        si = jnp.minimum(jnp.maximum(start, USER_N), N_NODES - CHUNK)
        return (
            start + CHUNK <= USER_N,
            pltpu.make_async_copy(user_hbm.at[pl.ds(start, CHUNK)], bufs[u],
                                  sem.at[u]),
            pltpu.make_async_copy(item_hbm.at[pl.ds(si - USER_N, CHUNK)], bufs[u],
                                  sem.at[u]),
            si,
        )

    def issue(m, u):
        is_user, du, di, _ = descs(m, u)

        @pl.when(is_user)
        def _():
            du.start()

        @pl.when(jnp.logical_not(is_user))
        def _():
            di.start()

    NCH = STRIPE // CHUNK + 1
    issue(jnp.int32(0), 0)

    def step(t, _):
        for u in range(2):
            m = t * 2 + u

            @pl.when(m < NCH)
            def _(m=m, u=u):
                @pl.when(m + 1 < NCH)
                def _():
                    issue(m + 1, 1 - u)
                is_user, du, _, si = descs(m, u)
                du.wait()
                start = start_of(m)
                wstart = jnp.where(is_user, start, si)
                pltpu.sync_copy(bufs[u].at[:, pl.ds(c * HALF, HALF)],
                                h0_hbm.at[c].at[pl.ds(wstart, CHUNK)])
                straddle = jnp.logical_and(start < USER_N, start + CHUNK > USER_N)

                @pl.when(straddle)
                def _():
                    pltpu.sync_copy(user_hbm.at[pl.ds(USER_N - CHUNK, CHUNK)], bx)
                    pltpu.sync_copy(bx.at[:, pl.ds(c * HALF, HALF)],
                                    h0_hbm.at[c].at[pl.ds(USER_N - CHUNK, CHUNK)])
        return 0
    lax.fori_loop(0, (NCH + 1) // 2, step, 0)


@jax.jit
def _split(user_emb, item_emb):
    return pl.kernel(
        _split_body,
        out_type=jax.ShapeDtypeStruct((NC, N_PAD, HALF), jnp.float32),
        mesh=plsc.VectorSubcoreMesh(core_axis_name="c", subcore_axis_name="s"),
        scratch_types=[
            pltpu.VMEM((CHUNK, DIM), jnp.float32),
            pltpu.VMEM((CHUNK, DIM), jnp.float32),
            pltpu.VMEM((CHUNK, DIM), jnp.float32),
            pltpu.SemaphoreType.DMA((2,)),
        ],
        compiler_params=pltpu.CompilerParams(use_tc_tiling_on_sc=False),
    )(user_emb, item_emb)


QROWS = N_NODES // 4
CB = 544


def _combine_body(a, b, e, o):
    x0 = (a[0] + b[0] + e[0]) * (1.0 / 3.0)
    x1 = (a[1] + b[1] + e[1]) * (1.0 / 3.0)
    o[:, 0, :] = jnp.concatenate(
        [x0[:, 0:32], x1[:, 0:32], x0[:, 32:64], x1[:, 32:64]], axis=1)
    o[:, 1, :] = jnp.concatenate(
        [x0[:, 64:96], x1[:, 64:96], x0[:, 96:128], x1[:, 96:128]], axis=1)


@jax.jit
def _combine(h0, h1, h2):
    spec = pl.BlockSpec((2, CB, 128), lambda i: (0, i, 0))
    out = pl.pallas_call(
        _combine_body,
        out_shape=jax.ShapeDtypeStruct((QROWS, 2, 128), jnp.float32),
        grid=(-(-QROWS // CB),),
        in_specs=[spec, spec, spec],
        out_specs=pl.BlockSpec((CB, 2, 128), lambda i: (i, 0, 0)),
    )(h0.reshape(NC, N_PAD // 4, 128), h1.reshape(NC, N_PAD // 4, 128),
      h2.reshape(NC, N_PAD // 4, 128))
    return out.reshape(N_NODES, DIM)


def kernel(edge_index, edge_weight, user_emb, item_emb):
    ei = edge_index.astype(jnp.int32).reshape(2, NBT, KCH, CHUNK)

    h0 = _split(user_emb, item_emb)
    h1 = _layer(ei, edge_weight, h0)
    h2 = _layer(ei, edge_weight, h1)
    return _combine(h0, h1, h2)
